# baseline (speedup 1.0000x reference)
import functools

import jax
import jax.numpy as jnp
from jax.experimental import pallas as pl
from jax.experimental.pallas import tpu as pltpu


_NSLOT = 4
_AHEAD = 2


def _gather_embed_kernel(ids_ref, word_hbm, pos_ref, out_ref, buf, sems, *,
                         tile, n_inner, seq_len):
    c = pl.program_id(0)
    j = pl.program_id(1)
    slot = j % _NSLOT

    def issue_rows(inner_idx):
        base = (c * n_inner + inner_idx) * tile
        s = inner_idx % _NSLOT
        for r in range(tile):
            row = ids_ref[base + r]
            pltpu.make_async_copy(word_hbm.at[pl.ds(row, 1)],
                                  buf.at[s, pl.ds(r, 1)],
                                  sems.at[s]).start()

    @pl.when(j == 0)
    def _():
        for k in range(_AHEAD):
            if k < n_inner:
                issue_rows(k)

    @pl.when(j + _AHEAD < n_inner)
    def _():
        issue_rows(j + _AHEAD)

    pltpu.make_async_copy(word_hbm.at[pl.ds(0, tile)], buf.at[slot],
                          sems.at[slot]).wait()

    for k in range(tile // seq_len):
        out_ref[pl.ds(k * seq_len, seq_len), :] = (
            buf[slot, pl.ds(k * seq_len, seq_len), :] + pos_ref[...])


def kernel(inputs, word_table, pos_table):
    B, L = inputs.shape
    V, D = word_table.shape
    S, D2 = pos_table.shape
    assert D == D2 and L <= S

    word_table = word_table.astype(jnp.float32)
    pos_table = pos_table.astype(jnp.float32)

    n_tokens = B * L
    tile = 2 * L if B % 4 == 0 else L
    n_tiles = n_tokens // tile
    n_cores = 2 if n_tiles % 2 == 0 else 1
    n_inner = n_tiles // n_cores

    ids = jnp.clip(inputs.astype(jnp.int32), 0, V - 1)
    ids_flat = ids.reshape(n_tokens)

    kernel_fn = functools.partial(_gather_embed_kernel, tile=tile,
                                  n_inner=n_inner, seq_len=L)
    out_flat = pl.pallas_call(
        kernel_fn,
        out_shape=jax.ShapeDtypeStruct((n_tokens, D), jnp.float32),
        grid_spec=pltpu.PrefetchScalarGridSpec(
            num_scalar_prefetch=1,
            grid=(n_cores, n_inner),
            in_specs=[
                pl.BlockSpec(memory_space=pl.ANY),
                pl.BlockSpec((L, D), lambda c, j, ids: (0, 0)),
            ],
            out_specs=pl.BlockSpec((tile, D),
                                   lambda c, j, ids: (c * n_inner + j, 0)),
            scratch_shapes=[
                pltpu.VMEM((_NSLOT, tile, D), jnp.float32),
                pltpu.SemaphoreType.DMA((_NSLOT,)),
            ],
        ),
        compiler_params=pltpu.CompilerParams(
            dimension_semantics=("parallel", "arbitrary"),
            vmem_limit_bytes=64 * 1024 * 1024),
    )(ids_flat, word_table, pos_table[:L])

    return out_flat.reshape(B, L, D)


# speedup vs baseline: 1.0013x; 1.0013x over previous
import functools

import jax
import jax.numpy as jnp
from jax.experimental import pallas as pl
from jax.experimental.pallas import tpu as pltpu


_NSLOT = 4
_AHEAD = 2
_NSUB = 4


def _gather_embed_kernel(ids_ref, word_hbm, pos_ref, out_ref, buf, sems, *,
                         tile, n_inner, seq_len):
    c = pl.program_id(0)
    j = pl.program_id(1)
    slot = j % _NSLOT
    sub = tile // _NSUB

    def issue_rows(inner_idx):
        base = (c * n_inner + inner_idx) * tile
        s = inner_idx % _NSLOT
        for r in range(tile):
            row = ids_ref[base + r]
            pltpu.make_async_copy(word_hbm.at[pl.ds(row, 1)],
                                  buf.at[s, pl.ds(r, 1)],
                                  sems.at[s, r // sub]).start()

    @pl.when(j == 0)
    def _():
        for k in range(_AHEAD):
            if k < n_inner:
                issue_rows(k)

    @pl.when(j + _AHEAD < n_inner)
    def _():
        issue_rows(j + _AHEAD)

    for k in range(_NSUB):
        pltpu.make_async_copy(word_hbm.at[pl.ds(0, sub)],
                              buf.at[slot, pl.ds(k * sub, sub)],
                              sems.at[slot, k]).wait()
        pos_base = (k * sub) % seq_len
        out_ref[pl.ds(k * sub, sub), :] = (
            buf[slot, pl.ds(k * sub, sub), :]
            + pos_ref[pl.ds(pos_base, sub), :])


def kernel(inputs, word_table, pos_table):
    B, L = inputs.shape
    V, D = word_table.shape
    S, D2 = pos_table.shape
    assert D == D2 and L <= S

    word_table = word_table.astype(jnp.float32)
    pos_table = pos_table.astype(jnp.float32)

    n_tokens = B * L
    tile = 2 * L if B % 4 == 0 else L
    n_tiles = n_tokens // tile
    n_cores = 2 if n_tiles % 2 == 0 else 1
    n_inner = n_tiles // n_cores

    ids = jnp.clip(inputs.astype(jnp.int32), 0, V - 1)
    ids_flat = ids.reshape(n_tokens)

    kernel_fn = functools.partial(_gather_embed_kernel, tile=tile,
                                  n_inner=n_inner, seq_len=L)
    out_flat = pl.pallas_call(
        kernel_fn,
        out_shape=jax.ShapeDtypeStruct((n_tokens, D), jnp.float32),
        grid_spec=pltpu.PrefetchScalarGridSpec(
            num_scalar_prefetch=1,
            grid=(n_cores, n_inner),
            in_specs=[
                pl.BlockSpec(memory_space=pl.ANY),
                pl.BlockSpec((L, D), lambda c, j, ids: (0, 0)),
            ],
            out_specs=pl.BlockSpec((tile, D),
                                   lambda c, j, ids: (c * n_inner + j, 0)),
            scratch_shapes=[
                pltpu.VMEM((_NSLOT, tile, D), jnp.float32),
                pltpu.SemaphoreType.DMA((_NSLOT, _NSUB)),
            ],
        ),
        compiler_params=pltpu.CompilerParams(
            dimension_semantics=("parallel", "arbitrary"),
            vmem_limit_bytes=64 * 1024 * 1024),
    )(ids_flat, word_table, pos_table[:L])

    return out_flat.reshape(B, L, D)
